# baseline (speedup 1.0000x reference)
import functools

import jax
import jax.numpy as jnp
from jax.experimental import pallas as pl
from jax.experimental.pallas import tpu as pltpu

_CB = 128
_G = 16


def _filter_dma_kernel(B, C, HW, ch_ref, x_ref, o_ref, zbuf, xb, csem, zsem, bsem):
    NC = C // _CB
    ch = ch_ref[0]
    kb = jax.lax.div(ch, _CB)
    rem = jax.lax.rem(ch, _CB)

    zbuf[...] = jnp.zeros_like(zbuf)

    for k in range(NC):
        @pl.when((k + 1) * _CB <= ch)
        def _(k=k):
            pltpu.make_async_copy(
                x_ref.at[:, k * _CB:(k + 1) * _CB, :],
                o_ref.at[:, k * _CB:(k + 1) * _CB, :],
                csem.at[k],
            ).start()

    for k in range(NC):
        for g in range(B // _G):
            @pl.when(k * _CB >= ch)
            def _(k=k, g=g):
                pltpu.make_async_copy(
                    zbuf,
                    o_ref.at[g * _G:(g + 1) * _G, k * _CB:(k + 1) * _CB, :],
                    zsem.at[k],
                ).start()

    @pl.when((rem != 0) & (ch < C))
    def _():
        cin = pltpu.make_async_copy(x_ref.at[:, pl.ds(kb * _CB, _CB), :], xb, bsem)
        cin.start()
        cin.wait()
        ci = jax.lax.broadcasted_iota(jnp.int32, xb.shape, 1) + kb * _CB
        xb[...] = jnp.where(ci < ch, xb[...], 0.0)
        cout = pltpu.make_async_copy(xb, o_ref.at[:, pl.ds(kb * _CB, _CB), :], bsem)
        cout.start()
        cout.wait()

    for k in range(NC):
        @pl.when((k + 1) * _CB <= ch)
        def _(k=k):
            pltpu.make_async_copy(
                x_ref.at[:, k * _CB:(k + 1) * _CB, :],
                o_ref.at[:, k * _CB:(k + 1) * _CB, :],
                csem.at[k],
            ).wait()

    for k in range(NC):
        for g in range(B // _G):
            @pl.when(k * _CB >= ch)
            def _(k=k, g=g):
                pltpu.make_async_copy(
                    zbuf,
                    o_ref.at[g * _G:(g + 1) * _G, k * _CB:(k + 1) * _CB, :],
                    zsem.at[k],
                ).wait()


def kernel(x, channels):
    B, C, H, W = x.shape
    HW = H * W
    x2 = x.reshape(B, C, HW)
    ch = jnp.asarray(channels, jnp.int32).reshape(1)
    out = pl.pallas_call(
        functools.partial(_filter_dma_kernel, B, C, HW),
        grid_spec=pltpu.PrefetchScalarGridSpec(
            num_scalar_prefetch=1,
            grid=(1,),
            in_specs=[pl.BlockSpec(memory_space=pltpu.MemorySpace.HBM)],
            out_specs=pl.BlockSpec(memory_space=pltpu.MemorySpace.HBM),
            scratch_shapes=[
                pltpu.VMEM((_G, _CB, HW), x.dtype),
                pltpu.VMEM((B, _CB, HW), x.dtype),
                pltpu.SemaphoreType.DMA((C // _CB,)),
                pltpu.SemaphoreType.DMA((C // _CB,)),
                pltpu.SemaphoreType.DMA,
            ],
        ),
        out_shape=jax.ShapeDtypeStruct((B, C, HW), x.dtype),
    )(ch, x2)
    return out.reshape(B, C, H, W)


# speedup vs baseline: 9.7293x; 9.7293x over previous
import functools

import jax
import jax.numpy as jnp
from jax.experimental import pallas as pl
from jax.experimental.pallas import tpu as pltpu

_S = 8
_CBLK = 256


def _filter_kernel(B, C, HW, ch_ref, x_ref, o_ref, xbuf, obuf, isem, osem):
    NT = C // _CBLK
    NU = B * NT
    NI = NU // _S
    ch = ch_ref[0]

    def unit(s, i):
        u = s * NI + i
        return jax.lax.div(u, NT), jax.lax.rem(u, NT)

    def in_copy(s, i, slot):
        b, t = unit(s, i)
        return pltpu.make_async_copy(
            x_ref.at[pl.ds(b, 1), pl.ds(t * _CBLK, _CBLK), :],
            xbuf.at[s, slot], isem.at[s, slot])

    def out_copy(s, i, slot):
        b, t = unit(s, i)
        return pltpu.make_async_copy(
            obuf.at[s, slot],
            o_ref.at[pl.ds(b, 1), pl.ds(t * _CBLK, _CBLK), :], osem.at[s, slot])

    def skip(s, i):
        _, t = unit(s, i)
        return t * _CBLK >= ch

    for s in range(_S):
        @pl.when(jnp.logical_not(skip(s, 0)))
        def _(s=s):
            in_copy(s, 0, 0).start()

    def body(i, carry):
        slot = jax.lax.rem(i, 2)
        nslot = jax.lax.rem(i + 1, 2)
        for s in range(_S):
            @pl.when((i + 1 < NI) & jnp.logical_not(skip(s, i + 1)))
            def _(s=s):
                in_copy(s, i + 1, nslot).start()
            @pl.when(i >= 2)
            def _(s=s):
                out_copy(s, i - 2, slot).wait()
            @pl.when(jnp.logical_not(skip(s, i)))
            def _(s=s):
                in_copy(s, i, slot).wait()
                _, t = unit(s, i)
                cidx = jax.lax.broadcasted_iota(
                    jnp.int32, (1, _CBLK, HW), 1) + t * _CBLK
                obuf[s, slot] = jnp.where(cidx < ch, xbuf[s, slot], 0.0)
            @pl.when(skip(s, i))
            def _(s=s):
                obuf[s, slot] = jnp.zeros((1, _CBLK, HW), xbuf.dtype)
            out_copy(s, i, slot).start()
        return carry

    jax.lax.fori_loop(0, NI, body, 0)

    for s in range(_S):
        out_copy(s, NI - 2, jax.lax.rem(NI - 2, 2)).wait()
        out_copy(s, NI - 1, jax.lax.rem(NI - 1, 2)).wait()


def kernel(x, channels):
    B, C, H, W = x.shape
    HW = H * W
    x2 = x.reshape(B, C, HW)
    ch = jnp.asarray(channels, jnp.int32).reshape(1)
    out = pl.pallas_call(
        functools.partial(_filter_kernel, B, C, HW),
        grid_spec=pltpu.PrefetchScalarGridSpec(
            num_scalar_prefetch=1,
            grid=(1,),
            in_specs=[pl.BlockSpec(memory_space=pltpu.MemorySpace.HBM)],
            out_specs=pl.BlockSpec(memory_space=pltpu.MemorySpace.HBM),
            scratch_shapes=[
                pltpu.VMEM((_S, 2, 1, _CBLK, HW), x.dtype),
                pltpu.VMEM((_S, 2, 1, _CBLK, HW), x.dtype),
                pltpu.SemaphoreType.DMA((_S, 2)),
                pltpu.SemaphoreType.DMA((_S, 2)),
            ],
        ),
        out_shape=jax.ShapeDtypeStruct((B, C, HW), x.dtype),
    )(ch, x2)
    return out.reshape(B, C, H, W)


# speedup vs baseline: 33.3010x; 3.4228x over previous
import jax
import jax.numpy as jnp
from jax.experimental import pallas as pl
from jax.experimental.pallas import tpu as pltpu


def _mask_mul_kernel(ch_ref, x_ref, o_ref):
    ch = ch_ref[0]
    c = jax.lax.broadcasted_iota(jnp.int32, x_ref.shape, 3)
    o_ref[...] = jnp.where(c < ch, x_ref[...], 0.0)


def kernel(x, channels):
    B, C, H, W = x.shape
    xt = jnp.transpose(x, (0, 2, 3, 1))
    ch = jnp.asarray(channels, jnp.int32).reshape(1)
    out = pl.pallas_call(
        _mask_mul_kernel,
        grid_spec=pltpu.PrefetchScalarGridSpec(
            num_scalar_prefetch=1,
            grid=(B,),
            in_specs=[pl.BlockSpec((1, H, W, C), lambda b, ch: (b, 0, 0, 0))],
            out_specs=pl.BlockSpec((1, H, W, C), lambda b, ch: (b, 0, 0, 0)),
        ),
        out_shape=jax.ShapeDtypeStruct((B, H, W, C), x.dtype),
    )(ch, xt)
    return jnp.transpose(out, (0, 3, 1, 2))


# speedup vs baseline: 35.6454x; 1.0704x over previous
import functools

import jax
import jax.numpy as jnp
from jax.experimental import pallas as pl
from jax.experimental.pallas import tpu as pltpu

_LB = 256


def _filter_kernel(B, C, H, W, ch_ref, x_ref, o_ref, xbuf, isem):
    NK = C // _LB
    ch = ch_ref[0]
    b = pl.program_id(0)

    def chunk_copy(bb, slot, k):
        return pltpu.make_async_copy(
            x_ref.at[pl.ds(bb, 1), :, :, pl.ds(k * _LB, _LB)],
            xbuf.at[slot, :, :, :, pl.ds(k * _LB, _LB)],
            isem.at[slot, k])

    @pl.when(b == 0)
    def _():
        for k in range(NK):
            @pl.when(k * _LB < ch)
            def _(k=k):
                chunk_copy(0, 0, k).start()

    @pl.when(b + 1 < B)
    def _():
        for k in range(NK):
            @pl.when(k * _LB < ch)
            def _(k=k):
                chunk_copy(b + 1, jax.lax.rem(b + 1, 2), k).start()

    slot = jax.lax.rem(b, 2)
    for k in range(NK):
        @pl.when(k * _LB < ch)
        def _(k=k):
            chunk_copy(b, slot, k).wait()

    c = jax.lax.broadcasted_iota(jnp.int32, (1, H, W, C), 3)
    o_ref[...] = jnp.where(c < ch, xbuf[slot], 0.0)


def kernel(x, channels):
    B, C, H, W = x.shape
    xt = jnp.transpose(x, (0, 2, 3, 1))
    ch = jnp.asarray(channels, jnp.int32).reshape(1)
    out = pl.pallas_call(
        functools.partial(_filter_kernel, B, C, H, W),
        grid_spec=pltpu.PrefetchScalarGridSpec(
            num_scalar_prefetch=1,
            grid=(B,),
            in_specs=[pl.BlockSpec(memory_space=pltpu.MemorySpace.HBM)],
            out_specs=pl.BlockSpec((1, H, W, C), lambda b, ch: (b, 0, 0, 0)),
            scratch_shapes=[
                pltpu.VMEM((2, 1, H, W, C), x.dtype),
                pltpu.SemaphoreType.DMA((2, C // _LB)),
            ],
        ),
        out_shape=jax.ShapeDtypeStruct((B, H, W, C), x.dtype),
    )(ch, xt)
    return jnp.transpose(out, (0, 3, 1, 2))


# speedup vs baseline: 43.2276x; 1.2127x over previous
import functools

import jax
import jax.numpy as jnp
from jax.experimental import pallas as pl
from jax.experimental.pallas import tpu as pltpu

_LB = 256
_BB = 2


def _filter_kernel(B, C, H, W, ch_ref, x_ref, o_ref, xbuf, isem):
    NK = C // _LB
    NB = B // _BB
    ch = ch_ref[0]
    i = pl.program_id(0)

    def chunk_copy(ii, slot, k):
        return pltpu.make_async_copy(
            x_ref.at[pl.ds(ii * _BB, _BB), :, :, pl.ds(k * _LB, _LB)],
            xbuf.at[slot, :, :, :, pl.ds(k * _LB, _LB)],
            isem.at[slot, k])

    @pl.when(i == 0)
    def _():
        for k in range(NK):
            @pl.when(k * _LB < ch)
            def _(k=k):
                chunk_copy(0, 0, k).start()

    @pl.when(i + 1 < NB)
    def _():
        for k in range(NK):
            @pl.when(k * _LB < ch)
            def _(k=k):
                chunk_copy(i + 1, jax.lax.rem(i + 1, 2), k).start()

    slot = jax.lax.rem(i, 2)
    for k in range(NK):
        @pl.when(k * _LB < ch)
        def _(k=k):
            chunk_copy(i, slot, k).wait()

    c = jax.lax.broadcasted_iota(jnp.int32, (_BB, H, W, C), 3)
    o_ref[...] = jnp.where(c < ch, xbuf[slot], 0.0)


def kernel(x, channels):
    B, C, H, W = x.shape
    xt = jnp.transpose(x, (0, 2, 3, 1))
    ch = jnp.asarray(channels, jnp.int32).reshape(1)
    out = pl.pallas_call(
        functools.partial(_filter_kernel, B, C, H, W),
        grid_spec=pltpu.PrefetchScalarGridSpec(
            num_scalar_prefetch=1,
            grid=(B // _BB,),
            in_specs=[pl.BlockSpec(memory_space=pltpu.MemorySpace.HBM)],
            out_specs=pl.BlockSpec((_BB, H, W, C), lambda i, ch: (i, 0, 0, 0)),
            scratch_shapes=[
                pltpu.VMEM((2, _BB, H, W, C), x.dtype),
                pltpu.SemaphoreType.DMA((2, C // _LB)),
            ],
        ),
        out_shape=jax.ShapeDtypeStruct((B, H, W, C), x.dtype),
    )(ch, xt)
    return jnp.transpose(out, (0, 3, 1, 2))


# speedup vs baseline: 45.3410x; 1.0489x over previous
import functools

import jax
import jax.numpy as jnp
from jax.experimental import pallas as pl
from jax.experimental.pallas import tpu as pltpu

_LB = 256
_BB = 4


def _filter_kernel(B, C, H, W, ch_ref, x_ref, o_ref, xbuf, isem):
    NK = C // _LB
    NB = B // _BB
    ch = ch_ref[0]
    i = pl.program_id(0)

    def chunk_copy(ii, slot, k):
        return pltpu.make_async_copy(
            x_ref.at[pl.ds(ii * _BB, _BB), :, :, pl.ds(k * _LB, _LB)],
            xbuf.at[slot, :, :, :, pl.ds(k * _LB, _LB)],
            isem.at[slot, k])

    @pl.when(i == 0)
    def _():
        for k in range(NK):
            @pl.when(k * _LB < ch)
            def _(k=k):
                chunk_copy(0, 0, k).start()

    @pl.when(i + 1 < NB)
    def _():
        for k in range(NK):
            @pl.when(k * _LB < ch)
            def _(k=k):
                chunk_copy(i + 1, jax.lax.rem(i + 1, 2), k).start()

    slot = jax.lax.rem(i, 2)
    for k in range(NK):
        @pl.when(k * _LB < ch)
        def _(k=k):
            chunk_copy(i, slot, k).wait()

    c = jax.lax.broadcasted_iota(jnp.int32, (_BB, H, W, C), 3)
    o_ref[...] = jnp.where(c < ch, xbuf[slot], 0.0)


def kernel(x, channels):
    B, C, H, W = x.shape
    xt = jnp.transpose(x, (0, 2, 3, 1))
    ch = jnp.asarray(channels, jnp.int32).reshape(1)
    out = pl.pallas_call(
        functools.partial(_filter_kernel, B, C, H, W),
        grid_spec=pltpu.PrefetchScalarGridSpec(
            num_scalar_prefetch=1,
            grid=(B // _BB,),
            in_specs=[pl.BlockSpec(memory_space=pltpu.MemorySpace.HBM)],
            out_specs=pl.BlockSpec((_BB, H, W, C), lambda i, ch: (i, 0, 0, 0)),
            scratch_shapes=[
                pltpu.VMEM((2, _BB, H, W, C), x.dtype),
                pltpu.SemaphoreType.DMA((2, C // _LB)),
            ],
        ),
        out_shape=jax.ShapeDtypeStruct((B, H, W, C), x.dtype),
    )(ch, xt)
    return jnp.transpose(out, (0, 3, 1, 2))


# speedup vs baseline: 47.2084x; 1.0412x over previous
import functools

import jax
import jax.numpy as jnp
from jax.experimental import pallas as pl
from jax.experimental.pallas import tpu as pltpu

_LB = 256
_BB = 8


def _filter_kernel(B, C, H, W, ch_ref, x_ref, o_ref, xbuf, isem):
    NK = C // _LB
    NB = B // _BB
    ch = ch_ref[0]
    i = pl.program_id(0)

    def chunk_copy(ii, slot, k):
        return pltpu.make_async_copy(
            x_ref.at[pl.ds(ii * _BB, _BB), :, :, pl.ds(k * _LB, _LB)],
            xbuf.at[slot, :, :, :, pl.ds(k * _LB, _LB)],
            isem.at[slot, k])

    @pl.when(i == 0)
    def _():
        for k in range(NK):
            @pl.when(k * _LB < ch)
            def _(k=k):
                chunk_copy(0, 0, k).start()

    @pl.when(i + 1 < NB)
    def _():
        for k in range(NK):
            @pl.when(k * _LB < ch)
            def _(k=k):
                chunk_copy(i + 1, jax.lax.rem(i + 1, 2), k).start()

    slot = jax.lax.rem(i, 2)
    for k in range(NK):
        @pl.when(k * _LB < ch)
        def _(k=k):
            chunk_copy(i, slot, k).wait()

    c = jax.lax.broadcasted_iota(jnp.int32, (_BB, H, W, C), 3)
    o_ref[...] = jnp.where(c < ch, xbuf[slot], 0.0)


def kernel(x, channels):
    B, C, H, W = x.shape
    xt = jnp.transpose(x, (0, 2, 3, 1))
    ch = jnp.asarray(channels, jnp.int32).reshape(1)
    out = pl.pallas_call(
        functools.partial(_filter_kernel, B, C, H, W),
        grid_spec=pltpu.PrefetchScalarGridSpec(
            num_scalar_prefetch=1,
            grid=(B // _BB,),
            in_specs=[pl.BlockSpec(memory_space=pltpu.MemorySpace.HBM)],
            out_specs=pl.BlockSpec((_BB, H, W, C), lambda i, ch: (i, 0, 0, 0)),
            scratch_shapes=[
                pltpu.VMEM((2, _BB, H, W, C), x.dtype),
                pltpu.SemaphoreType.DMA((2, C // _LB)),
            ],
        ),
        out_shape=jax.ShapeDtypeStruct((B, H, W, C), x.dtype),
    )(ch, xt)
    return jnp.transpose(out, (0, 3, 1, 2))


# speedup vs baseline: 47.2530x; 1.0009x over previous
import functools

import jax
import jax.numpy as jnp
from jax.experimental import pallas as pl
from jax.experimental.pallas import tpu as pltpu

_CUTS = (0, 512, 640, 768)
_BB = 8


def _filter_kernel(B, C, H, W, ch_ref, x_ref, o_ref, xbuf, isem):
    NK = len(_CUTS) - 1
    NB = B // _BB
    ch = ch_ref[0]
    i = pl.program_id(0)

    def chunk_copy(ii, slot, k):
        lo, hi = _CUTS[k], _CUTS[k + 1]
        return pltpu.make_async_copy(
            x_ref.at[pl.ds(ii * _BB, _BB), :, :, pl.ds(lo, hi - lo)],
            xbuf.at[slot, :, :, :, pl.ds(lo, hi - lo)],
            isem.at[slot, k])

    @pl.when(i == 0)
    def _():
        for k in range(NK):
            @pl.when(_CUTS[k] < ch)
            def _(k=k):
                chunk_copy(0, 0, k).start()

    @pl.when(i + 1 < NB)
    def _():
        for k in range(NK):
            @pl.when(_CUTS[k] < ch)
            def _(k=k):
                chunk_copy(i + 1, jax.lax.rem(i + 1, 2), k).start()

    slot = jax.lax.rem(i, 2)
    for k in range(NK):
        @pl.when(_CUTS[k] < ch)
        def _(k=k):
            chunk_copy(i, slot, k).wait()

    c = jax.lax.broadcasted_iota(jnp.int32, (_BB, H, W, C), 3)
    o_ref[...] = jnp.where(c < ch, xbuf[slot], 0.0)


def kernel(x, channels):
    B, C, H, W = x.shape
    xt = jnp.transpose(x, (0, 2, 3, 1))
    ch = jnp.asarray(channels, jnp.int32).reshape(1)
    out = pl.pallas_call(
        functools.partial(_filter_kernel, B, C, H, W),
        grid_spec=pltpu.PrefetchScalarGridSpec(
            num_scalar_prefetch=1,
            grid=(B // _BB,),
            in_specs=[pl.BlockSpec(memory_space=pltpu.MemorySpace.HBM)],
            out_specs=pl.BlockSpec((_BB, H, W, C), lambda i, ch: (i, 0, 0, 0)),
            scratch_shapes=[
                pltpu.VMEM((2, _BB, H, W, C), x.dtype),
                pltpu.SemaphoreType.DMA((2, len(_CUTS) - 1)),
            ],
        ),
        out_shape=jax.ShapeDtypeStruct((B, H, W, C), x.dtype),
    )(ch, xt)
    return jnp.transpose(out, (0, 3, 1, 2))
